# Initial kernel scaffold; baseline (speedup 1.0000x reference)
import jax
import jax.numpy as jnp
from jax.experimental import pallas as pl


def kernel(edges, features, W0, b0, W1, b1, W2, b2):
    raise NotImplementedError("write your pallas kernel here")


# speedup vs baseline: 8.4719x; 8.4719x over previous
import functools

import jax
import jax.numpy as jnp
from jax import lax
from jax.experimental import pallas as pl
from jax.experimental.pallas import tpu as pltpu
from jax.experimental.pallas import tpu_sc as plsc

N = 10000
E = 320000
HID = 64
HE = 128
NC = 2
NS = 16
NW = NC * NS
B = 128
CHUNKS = -(-E // (B * NW))
E_PAD = CHUNKS * B * NW
TRASH = N
ACC_ROWS = 10240
ZCH = ACC_ROWS // (NS * B)
OUT_PS = ACC_ROWS // NS


@functools.cache
def _mesh():
    return plsc.VectorSubcoreMesh(
        core_axis_name="c", subcore_axis_name="s", num_cores=NC, num_subcores=NS)


def _deg_body(col_hbm, ones_hbm, zeros_hbm, out_hbm, col_v, ones_v, acc):
    c = lax.axis_index("c")
    s = lax.axis_index("s")
    w = s * NC + c
    pltpu.sync_copy(zeros_hbm, ones_v)
    for k in range(ZCH):
        pltpu.sync_copy(ones_v, acc.at[pl.ds((s * ZCH + k) * B, B)])
    pltpu.sync_copy(ones_hbm, ones_v)
    plsc.subcore_barrier()

    def body(j, carry):
        base = (w * CHUNKS + j) * B
        pltpu.sync_copy(col_hbm.at[pl.ds(base, B)], col_v)
        pltpu.sync_copy(ones_v, acc.at[col_v], add=True)
        return carry

    lax.fori_loop(0, CHUNKS, body, 0)
    plsc.subcore_barrier()
    pltpu.sync_copy(acc.at[pl.ds(s * OUT_PS, OUT_PS)],
                    out_hbm.at[c, pl.ds(s * OUT_PS, OUT_PS)])


@functools.cache
def _deg_call():
    return pl.kernel(
        _deg_body,
        out_type=jax.ShapeDtypeStruct((NC, ACC_ROWS, HE), jnp.float32),
        mesh=_mesh(),
        scratch_types=[
            pltpu.VMEM((B,), jnp.int32),
            pltpu.VMEM((B, HE), jnp.float32),
            pltpu.VMEM_SHARED((ACC_ROWS, HE), jnp.float32),
        ],
    )


def _edge_body(row_hbm, col_hbm, g_hbm, zeros_hbm, out_hbm,
               row_v, col_v, buf, acc, sem):
    c = lax.axis_index("c")
    s = lax.axis_index("s")
    pltpu.sync_copy(zeros_hbm, buf)
    for k in range(ZCH):
        pltpu.sync_copy(buf, acc.at[pl.ds((s * ZCH + k) * B, B)])
    plsc.subcore_barrier()

    base = (s * NC + c) * CHUNKS
    nch = CHUNKS

    def body(j, carry):
        o = (base + j) * B
        pltpu.sync_copy(row_hbm.at[pl.ds(o, B)], row_v)
        pltpu.sync_copy(col_hbm.at[pl.ds(o, B)], col_v)
        pltpu.async_copy(g_hbm.at[row_v], buf, sem).wait()
        pltpu.sync_copy(buf, acc.at[col_v], add=True)
        return carry

    lax.fori_loop(0, nch, body, 0)
    plsc.subcore_barrier()
    pltpu.sync_copy(acc.at[pl.ds(s * OUT_PS, OUT_PS)],
                    out_hbm.at[c, pl.ds(s * OUT_PS, OUT_PS)])


@functools.cache
def _edge_call():
    return pl.kernel(
        _edge_body,
        out_type=jax.ShapeDtypeStruct((NC, ACC_ROWS, HE), jnp.float32),
        mesh=_mesh(),
        scratch_types=[
            pltpu.VMEM((B,), jnp.int32),
            pltpu.VMEM((B,), jnp.int32),
            pltpu.VMEM((B, HE), jnp.float32),
            pltpu.VMEM_SHARED((ACC_ROWS, HE), jnp.float32),
            pltpu.SemaphoreType.DMA,
        ],
    )


R = 1000
G = N // R


def _dinv(degT_ref):
    deg = jnp.sum(degT_ref[...], axis=1, keepdims=True) + 1.0
    return lax.rsqrt(deg)


def _pad_he(x):
    return jnp.concatenate([x, jnp.zeros((x.shape[0], HE - HID), x.dtype)], axis=1)


def _first_body(x_ref, w_ref, degT_ref, o_ref):
    dinv = _dinv(degT_ref)
    h = jnp.dot(x_ref[...], w_ref[...], preferred_element_type=jnp.float32)
    o_ref[...] = _pad_he(dinv * h)


def _mid_body(sp_ref, g_ref, degT_ref, b_ref, w_ref, o_ref):
    dinv = _dinv(degT_ref)
    ssum = (sp_ref[0] + sp_ref[1] + g_ref[...])[:, :HID]
    x = jax.nn.relu(dinv * ssum + b_ref[...])
    o_ref[...] = _pad_he(
        dinv * jnp.dot(x, w_ref[...], preferred_element_type=jnp.float32))


def _preagg_body(sp_ref, g_ref, degT_ref, b_ref, o_ref):
    dinv = _dinv(degT_ref)
    ssum = (sp_ref[0] + sp_ref[1] + g_ref[...])[:, :HID]
    o_ref[...] = _pad_he(dinv * jax.nn.relu(dinv * ssum + b_ref[...]))


def _last_body(sp_ref, g_ref, degT_ref, w_ref, b_ref, o_ref):
    dinv = _dinv(degT_ref)
    agg = dinv * (sp_ref[0] + sp_ref[1] + g_ref[...])[:, :HID]
    h = jnp.dot(agg, w_ref[...], preferred_element_type=jnp.float32) + b_ref[...]
    m = jnp.max(h, axis=1, keepdims=True)
    lse = m + jnp.log(jnp.sum(jnp.exp(h - m), axis=1, keepdims=True))
    o_ref[...] = h - lse


def _rows(d):
    return pl.BlockSpec((R, d), lambda i: (i, 0))


def _whole(*shape):
    return pl.BlockSpec(shape, lambda i: (0,) * len(shape))


def _sp_spec():
    return pl.BlockSpec((NC, R, HE), lambda i: (0, i, 0))


def _tc_call(body, in_specs, out_dim):
    return pl.pallas_call(
        body,
        grid=(G,),
        in_specs=in_specs,
        out_specs=_rows(out_dim),
        out_shape=jax.ShapeDtypeStruct((N, out_dim), jnp.float32),
    )


def kernel(edges, features, W0, b0, W1, b1, W2, b2):
    row = edges[0]
    col = edges[1]
    pad = E_PAD - E
    row_p = jnp.concatenate([row, jnp.zeros((pad,), row.dtype)])
    col_p = jnp.concatenate([col, jnp.full((pad,), TRASH, col.dtype)])
    zeros_h = jnp.zeros((B, HE), jnp.float32)
    ones_h = jnp.ones((B, HE), jnp.float32)

    degp = _deg_call()(col_p, ones_h, zeros_h)
    degT = degp[:, :N, 0].T

    b0r = b0.reshape(1, HID)
    b1r = b1.reshape(1, HID)
    b2r = b2.reshape(1, W2.shape[1])

    g0 = _tc_call(
        _first_body,
        [_rows(features.shape[1]), _whole(features.shape[1], HID), _rows(2)],
        HE,
    )(features, W0, degT)

    s0 = _edge_call()(row_p, col_p, g0, zeros_h)

    g1 = _tc_call(
        _mid_body,
        [_sp_spec(), _rows(HE), _rows(2), _whole(1, HID), _whole(HID, HID)],
        HE,
    )(s0, g0, degT, b0r, W1)

    s1 = _edge_call()(row_p, col_p, g1, zeros_h)

    g2 = _tc_call(
        _preagg_body,
        [_sp_spec(), _rows(HE), _rows(2), _whole(1, HID)],
        HE,
    )(s1, g1, degT, b1r)

    s2 = _edge_call()(row_p, col_p, g2, zeros_h)

    d_out = W2.shape[1]
    out = _tc_call(
        _last_body,
        [_sp_spec(), _rows(HE), _rows(2), _whole(HID, d_out), _whole(1, d_out)],
        d_out,
    )(s2, g2, degT, W2, b2r)

    return out
